# baseline (speedup 1.0000x reference)
import functools

import jax
import jax.numpy as jnp
from jax import lax
from jax.experimental import pallas as pl
from jax.experimental.pallas import tpu as pltpu

BP = 128


def _fused_body(x_ref, batch_ref, mask_ref,
                W1_ref, b1_ref, W2_ref, b2_ref, w3_ref, b3_ref,
                vW1_ref, vb1_ref, vW2_ref, vb2_ref, vw3_ref, vb3_ref,
                z_ref, v_ref, sums_ref, counts_ref, *, nb):
    i = pl.program_id(0)

    @pl.when(i == 0)
    def _init():
        sums_ref[...] = jnp.zeros_like(sums_ref)
        counts_ref[...] = jnp.zeros_like(counts_ref)

    xb = x_ref[...]
    h = jnp.maximum(
        lax.dot_general(xb, W1_ref[...], (((1,), (1,)), ((), ())),
                        preferred_element_type=jnp.float32) + b1_ref[...], 0.0)
    h = jnp.maximum(
        lax.dot_general(h, W2_ref[...], (((1,), (1,)), ((), ())),
                        preferred_element_type=jnp.float32) + b2_ref[...], 0.0)
    z = lax.dot_general(w3_ref[...], h, (((1,), (1,)), ((), ())),
                        preferred_element_type=jnp.float32) + b3_ref[...]
    z = jnp.where(mask_ref[0] != 0, z, -jnp.inf)
    z_ref[...] = z[None]

    bvec = batch_ref[0]
    iota = lax.broadcasted_iota(jnp.int32, (BP, bvec.shape[1]), 0)
    ohT = (bvec == iota).astype(jnp.float32)
    sums_ref[...] += lax.dot_general(ohT, xb, (((1,), (0,)), ((), ())),
                                     preferred_element_type=jnp.float32)
    counts_ref[...] += jnp.sum(ohT, axis=1, keepdims=True)

    @pl.when(i == nb - 1)
    def _final():
        mean = sums_ref[...] / jnp.maximum(counts_ref[...], 1.0)
        hv = jnp.maximum(
            lax.dot_general(mean, vW1_ref[...], (((1,), (1,)), ((), ())),
                            preferred_element_type=jnp.float32) + vb1_ref[...], 0.0)
        hv = jnp.maximum(
            lax.dot_general(hv, vW2_ref[...], (((1,), (1,)), ((), ())),
                            preferred_element_type=jnp.float32) + vb2_ref[...], 0.0)
        v_ref[...] = jnp.sum(hv * vw3_ref[...], axis=1, keepdims=True) + vb3_ref[...]


def kernel(x, node_type, action_mask, node_indices, batch, N,
           mlp_W1, mlp_b1, mlp_W2, mlp_b2, mlp_W3, mlp_b3,
           vf_W1, vf_b1, vf_W2, vf_b2, vf_W3, vf_b3):
    n_nodes, d = x.shape
    b = N.shape[0]
    blk = 10000
    nb = n_nodes // blk

    batch3 = batch.astype(jnp.int32).reshape(nb, 1, blk)
    mask3 = action_mask.astype(jnp.int32).reshape(nb, 1, blk)
    b1r = mlp_b1.reshape(1, -1)
    b2r = mlp_b2.reshape(1, -1)
    w3r = mlp_W3.reshape(1, -1)
    b3r = mlp_b3.reshape(1, 1)
    vb1r = vf_b1.reshape(1, -1)
    vb2r = vf_b2.reshape(1, -1)
    vw3r = vf_W3.reshape(1, -1)
    vb3r = vf_b3.reshape(1, 1)

    full = lambda shape: pl.BlockSpec(shape, lambda i: (0,) * len(shape))
    z_lane, v_full = pl.pallas_call(
        functools.partial(_fused_body, nb=nb),
        grid=(nb,),
        in_specs=[
            pl.BlockSpec((blk, d), lambda i: (i, 0)),
            pl.BlockSpec((1, 1, blk), lambda i: (i, 0, 0)),
            pl.BlockSpec((1, 1, blk), lambda i: (i, 0, 0)),
            full(mlp_W1.shape), full(b1r.shape),
            full(mlp_W2.shape), full(b2r.shape),
            full(w3r.shape), full(b3r.shape),
            full(vf_W1.shape), full(vb1r.shape),
            full(vf_W2.shape), full(vb2r.shape),
            full(vw3r.shape), full(vb3r.shape),
        ],
        out_specs=[
            pl.BlockSpec((1, 1, blk), lambda i: (i, 0, 0)),
            pl.BlockSpec((BP, 1), lambda i: (0, 0)),
        ],
        out_shape=[
            jax.ShapeDtypeStruct((nb, 1, blk), jnp.float32),
            jax.ShapeDtypeStruct((BP, 1), jnp.float32),
        ],
        scratch_shapes=[
            pltpu.VMEM((BP, d), jnp.float32),
            pltpu.VMEM((BP, 1), jnp.float32),
        ],
        compiler_params=pltpu.CompilerParams(
            dimension_semantics=("arbitrary",)),
    )(x, batch3, mask3,
      mlp_W1, b1r, mlp_W2, b2r, w3r, b3r,
      vf_W1, vb1r, vf_W2, vb2r, vw3r, vb3r)
    return (z_lane.reshape(n_nodes, 1), v_full[:b])
